# baseline (speedup 1.0000x reference)
import functools

import jax
import jax.numpy as jnp
from jax import lax
from jax.experimental import pallas as pl
from jax.experimental.pallas import tpu as pltpu
from jax.experimental.pallas import tpu_sc as plsc

_NUM_EMB = 8192
_DIM = 256
_COMMIT = 0.25
_T = 1024


def _dist_argmin_kernel(x_ref, e_ref, idx_ref, loss_ref, b_ref):
    i = pl.program_id(0)

    @pl.when(i == 0)
    def _():
        e0 = e_ref[...]
        b_ref[...] = lax.dot_general(
            jnp.ones((1, _DIM), jnp.float32), e0 * e0,
            (((1,), (1,)), ((), ())))
        loss_ref[0, 0] = 0.0

    x = x_ref[...]
    e = e_ref[...]

    a = jnp.sum(x * x, axis=1, keepdims=True)
    m = lax.dot_general(x, e, (((1,), (1,)), ((), ())))

    nchunk = _NUM_EMB // 128
    mv = a + b_ref[0:1, 0:128] - 2.0 * m[:, 0:128]
    mif = jnp.zeros(mv.shape, jnp.float32)
    for c in range(1, nchunk):
        cs = slice(c * 128, (c + 1) * 128)
        blk = a + b_ref[0:1, cs] - 2.0 * m[:, cs]
        lt = blk < mv
        mif = jnp.where(lt, float(c), mif)
        mv = jnp.minimum(mv, blk)
    dmin = jnp.min(mv, axis=1, keepdims=True)
    lane = lax.broadcasted_iota(jnp.int32, mv.shape, 1).astype(jnp.float32)
    idxf = jnp.min(
        jnp.where(mv == dmin, mif * 128.0 + lane, float(_NUM_EMB)), axis=1)
    idx_ref[...] = idxf.astype(jnp.int32)[None, None, :]
    loss_ref[0, 0] += jnp.sum(dmin)


@functools.lru_cache(maxsize=1)
def _make_sc_gather():
    info = plsc.get_sparse_core_info()
    n_cores = info.num_cores
    rows_per_w = _NUM_EMB // (n_cores * info.num_subcores)

    def _sc_gather_kernel(table_hbm, idx_hbm, out_hbm, idx_v, rows_v, sem):
        wid = lax.axis_index("s") * n_cores + lax.axis_index("c")
        base = wid * rows_per_w
        pltpu.sync_copy(idx_hbm.at[pl.ds(base, rows_per_w)], idx_v)
        pltpu.async_copy(table_hbm.at[idx_v], rows_v, sem).wait()
        pltpu.sync_copy(rows_v, out_hbm.at[pl.ds(base, rows_per_w)])

    return functools.partial(
        pl.kernel,
        out_type=jax.ShapeDtypeStruct((_NUM_EMB, _DIM), jnp.float32),
        mesh=plsc.VectorSubcoreMesh(core_axis_name="c", subcore_axis_name="s"),
        scratch_types=[
            pltpu.VMEM((rows_per_w,), jnp.int32),
            pltpu.VMEM((rows_per_w, _DIM), jnp.float32),
            pltpu.SemaphoreType.DMA,
        ],
    )(_sc_gather_kernel)


def kernel(inputs, emb_weight):
    B, C, H, W = inputs.shape
    n_tok = B * H * W
    flat = jnp.transpose(inputs, (0, 2, 3, 1)).reshape(n_tok, _DIM)
    grid = (n_tok // _T,)

    idx3, loss_sum = pl.pallas_call(
        _dist_argmin_kernel,
        grid=grid,
        in_specs=[
            pl.BlockSpec((_T, _DIM), lambda i: (i, 0)),
            pl.BlockSpec((_NUM_EMB, _DIM), lambda i: (0, 0)),
        ],
        out_specs=[
            pl.BlockSpec((1, 1, _T), lambda i: (i, 0, 0)),
            pl.BlockSpec(memory_space=pltpu.SMEM),
        ],
        out_shape=[
            jax.ShapeDtypeStruct((n_tok // _T, 1, _T), jnp.int32),
            jax.ShapeDtypeStruct((1, 1), jnp.float32),
        ],
        scratch_shapes=[pltpu.VMEM((1, _NUM_EMB), jnp.float32)],
    )(flat, emb_weight)

    idx_flat = idx3.reshape(n_tok)
    q = _make_sc_gather()(emb_weight, idx_flat)

    encoding_indices = idx_flat.reshape(n_tok, 1)
    quantized_st = jnp.transpose(q.reshape(B, H, W, C), (0, 3, 1, 2))
    loss = (1.0 + _COMMIT) * loss_sum[0, 0] / (B * C * H * W)
    return (quantized_st, loss, encoding_indices)
